# Initial kernel scaffold; baseline (speedup 1.0000x reference)
import jax
import jax.numpy as jnp
from jax.experimental import pallas as pl


def kernel(x):
    raise NotImplementedError("write your pallas kernel here")


# speedup vs baseline: 2.4529x; 2.4529x over previous
import jax
import jax.numpy as jnp
from jax.experimental import pallas as pl

KEEP = 2048


def _body(x_ref, o_ref):
    xb = x_ref[0]
    C, H, W = xb.shape
    s = jnp.sum(jnp.abs(xb), axis=0)
    sb = jax.lax.bitcast_convert_type(s, jnp.int32)

    def vstep(i, t):
        cand = t + ((1 << 30) >> i)
        cnt = jnp.sum((sb >= cand).astype(jnp.int32))
        return jnp.where(cnt >= KEEP, cand, t)

    t = jax.lax.fori_loop(0, 31, vstep, jnp.int32(0))

    gt = sb > t
    eq = sb == t
    need = KEEP - jnp.sum(gt.astype(jnp.int32))
    row = jax.lax.broadcasted_iota(jnp.int32, (H, W), 0)
    col = jax.lax.broadcasted_iota(jnp.int32, (H, W), 1)
    nidx = row * W + col

    def istep(i, lo):
        cand = lo + ((1 << 13) >> i)
        cnt = jnp.sum((eq & (nidx <= cand)).astype(jnp.int32))
        return jnp.where(cnt < need, cand, lo)

    lo = jax.lax.fori_loop(0, 14, istep, jnp.int32(-1))
    m = (gt | (eq & (nidx <= lo + 1))).astype(jnp.float32)

    o_ref[0, 0] = jnp.sum(xb * m[None], axis=(1, 2)) * (1.0 / KEEP)


def kernel(x):
    B, C, H, W = x.shape
    out = pl.pallas_call(
        _body,
        grid=(B,),
        in_specs=[pl.BlockSpec((1, C, H, W), lambda b: (b, 0, 0, 0))],
        out_specs=pl.BlockSpec((1, 1, C), lambda b: (b, 0, 0)),
        out_shape=jax.ShapeDtypeStruct((B, 1, C), jnp.float32),
    )(x)
    return out[:, 0, :]


# speedup vs baseline: 4.6500x; 1.8957x over previous
import jax
import jax.numpy as jnp
from jax.experimental import pallas as pl

KEEP = 2048
G = 2


def _body(x_ref, o_ref):
    xb = x_ref[...]
    g, C, H, W = xb.shape
    s = jnp.sum(jnp.abs(xb), axis=1)
    sb = jax.lax.bitcast_convert_type(s, jnp.int32)

    def vstep(i, t):
        cand = t + ((1 << 30) >> i)
        cnt = jnp.sum((sb >= cand[:, None, None]).astype(jnp.int32), axis=(1, 2))
        return jnp.where(cnt >= KEEP, cand, t)

    t = jax.lax.fori_loop(0, 31, vstep, jnp.zeros((g,), jnp.int32))

    gt = sb > t[:, None, None]
    eq = sb == t[:, None, None]
    need = KEEP - jnp.sum(gt.astype(jnp.int32), axis=(1, 2))
    row = jax.lax.broadcasted_iota(jnp.int32, (H, W), 0)
    col = jax.lax.broadcasted_iota(jnp.int32, (H, W), 1)
    nidx = (row * W + col)[None]

    def istep(i, lo):
        cand = lo + ((1 << 13) >> i)
        cnt = jnp.sum((eq & (nidx <= cand[:, None, None])).astype(jnp.int32),
                      axis=(1, 2))
        return jnp.where(cnt < need, cand, lo)

    lo = jax.lax.fori_loop(0, 14, istep, jnp.full((g,), -1, jnp.int32))
    m = (gt | (eq & (nidx <= (lo + 1)[:, None, None]))).astype(jnp.float32)

    o_ref[...] = jnp.sum(xb * m[:, None], axis=(2, 3))[:, None, :] * (1.0 / KEEP)


def kernel(x):
    B, C, H, W = x.shape
    out = pl.pallas_call(
        _body,
        grid=(B // G,),
        in_specs=[pl.BlockSpec((G, C, H, W), lambda b: (b, 0, 0, 0))],
        out_specs=pl.BlockSpec((G, 1, C), lambda b: (b, 0, 0)),
        out_shape=jax.ShapeDtypeStruct((B, 1, C), jnp.float32),
    )(x)
    return out[:, 0, :]
